# baseline (speedup 1.0000x reference)
import functools

import jax
import jax.numpy as jnp
from jax import lax
from jax.experimental import pallas as pl
from jax.experimental.pallas import tpu as pltpu
from jax.experimental.pallas import tpu_sc as plsc

N = 50000
E = 800000
F = 64
H = 64
B = 512

NC = 2
NS = 16
NW = NC * NS

TW = 72
CK = 128
NCH_E = E // CK
CKN = 80
NCH_N = N // CKN
NPT = N // NS
WA = 32
WD = 8

_f32 = jnp.float32

_SC_LINEAR = pltpu.CompilerParams(use_tc_tiling_on_sc=False)


def _mesh():
    return plsc.VectorSubcoreMesh(core_axis_name="c", subcore_axis_name="s")


def _sc_gather(tabR, tabC, row2d, col2d):
    DEPTH = 2
    scratch = []
    for _ in range(DEPTH):
        scratch += [
            pltpu.VMEM((CK,), jnp.int32),
            pltpu.VMEM((CK,), jnp.int32),
            pltpu.VMEM((CK, TW), _f32),
            pltpu.VMEM((CK, TW), _f32),
            pltpu.SemaphoreType.DMA,
            pltpu.SemaphoreType.DMA,
        ]

    @functools.partial(
        pl.kernel,
        out_type=(
            jax.ShapeDtypeStruct((E, TW), _f32),
            jax.ShapeDtypeStruct((E, TW), _f32),
        ),
        mesh=_mesh(),
        scratch_types=scratch,
        compiler_params=_SC_LINEAR,
    )
    def k(tabR_h, tabC_h, row_h, col_h, gr_h, gc_h, *s):
        w = lax.axis_index("s") * NC + lax.axis_index("c")
        idxR = s[0::6]
        idxC = s[1::6]
        bufR = s[2::6]
        bufC = s[3::6]
        semI = s[4::6]
        semG = s[5::6]
        nsup = (NCH_E + NW * DEPTH - 1) // (NW * DEPTH)

        @pl.loop(0, nsup)
        def _(j):
            base = w + j * (NW * DEPTH)
            for b in range(DEPTH):
                c = base + b * NW

                @pl.when(c < NCH_E)
                def _():
                    pltpu.async_copy(row_h.at[c], idxR[b], semI[b])
                    pltpu.async_copy(col_h.at[c], idxC[b], semI[b])

            for b in range(DEPTH):
                c = base + b * NW

                @pl.when(c < NCH_E)
                def _():
                    pltpu.make_async_copy(row_h.at[c], idxR[b], semI[b]).wait()
                    pltpu.make_async_copy(col_h.at[c], idxC[b], semI[b]).wait()
                    pltpu.async_copy(tabR_h.at[idxR[b]], bufR[b], semG[b])
                    pltpu.async_copy(tabC_h.at[idxC[b]], bufC[b], semG[b])

            for b in range(DEPTH):
                c = base + b * NW

                @pl.when(c < NCH_E)
                def _():
                    pltpu.make_async_copy(
                        tabR_h.at[idxR[b]], bufR[b], semG[b]).wait()
                    pltpu.make_async_copy(
                        tabC_h.at[idxC[b]], bufC[b], semG[b]).wait()
                    pltpu.sync_copy(bufR[b], gr_h.at[pl.ds(c * CK, CK)])
                    pltpu.sync_copy(bufC[b], gc_h.at[pl.ds(c * CK, CK)])

    return k(tabR, tabC, row2d, col2d)


def _sc_scatter_m(row2d, vv, z32):
    DEPTH = 4
    scratch = [pltpu.VMEM_SHARED((N, WA), _f32)]
    for _ in range(DEPTH):
        scratch += [
            pltpu.VMEM((CK,), jnp.int32),
            pltpu.VMEM((CK, WA), _f32),
            pltpu.SemaphoreType.DMA,
        ]

    @functools.partial(
        pl.kernel,
        out_type=(
            jax.ShapeDtypeStruct((N, WA), _f32),
            jax.ShapeDtypeStruct((N, WA), _f32),
        ),
        mesh=_mesh(),
        scratch_types=scratch,
        compiler_params=_SC_LINEAR,
    )
    def k(row_h, vv_h, z32_h, a1_h, a2_h, *s):
        core = lax.axis_index("c")
        t = lax.axis_index("s")
        acc = s[0]
        idx = s[1::3]
        vbuf = s[2::3]
        sem = s[3::3]
        sl = pl.ds(t * NPT, NPT)

        pltpu.sync_copy(z32_h, acc.at[sl])
        plsc.subcore_barrier()

        def scan(coff):
            nsup = (NCH_E + NS * DEPTH - 1) // (NS * DEPTH)

            @pl.loop(0, nsup)
            def _(j):
                base = t + NS * DEPTH * j
                for b in range(DEPTH):
                    c = base + NS * b

                    @pl.when(c < NCH_E)
                    def _():
                        pltpu.async_copy(row_h.at[c], idx[b], sem[b])
                        pltpu.async_copy(
                            vv_h.at[pl.ds(c * CK, CK), pl.ds(coff, WA)],
                            vbuf[b], sem[b])

                for b in range(DEPTH):
                    c = base + NS * b

                    @pl.when(c < NCH_E)
                    def _():
                        pltpu.make_async_copy(
                            row_h.at[c], idx[b], sem[b]).wait()
                        pltpu.make_async_copy(
                            vv_h.at[pl.ds(c * CK, CK), pl.ds(coff, WA)],
                            vbuf[b], sem[b]).wait()
                        pltpu.sync_copy(vbuf[b], acc.at[idx[b]], add=True)

        @pl.when(core == 0)
        def _():
            scan(0)

        @pl.when(core == 1)
        def _():
            scan(WA)

        plsc.subcore_barrier()

        @pl.when(core == 0)
        def _():
            pltpu.sync_copy(acc.at[sl], a1_h.at[sl])

        @pl.when(core == 1)
        def _():
            pltpu.sync_copy(acc.at[sl], a2_h.at[sl])

    return k(row2d, vv, z32)


def _sc_scatter_t(row2d, vv, z8):
    DEPTH = 4
    scratch = [pltpu.VMEM_SHARED((N, WD), _f32)]
    for _ in range(DEPTH):
        scratch += [
            pltpu.VMEM((CK,), jnp.int32),
            pltpu.VMEM((CK, WD), _f32),
            pltpu.SemaphoreType.DMA,
        ]

    @functools.partial(
        pl.kernel,
        out_type=(
            jax.ShapeDtypeStruct((N, WD), _f32),
            jax.ShapeDtypeStruct((N, WD), _f32),
        ),
        mesh=_mesh(),
        scratch_types=scratch,
        compiler_params=_SC_LINEAR,
    )
    def k(row_h, vv_h, z8_h, d1_h, d2_h, *s):
        core = lax.axis_index("c")
        t = lax.axis_index("s")
        acc = s[0]
        idx = s[1::3]
        vbuf = s[2::3]
        sem = s[3::3]
        sl = pl.ds(t * NPT, NPT)

        pltpu.sync_copy(z8_h, acc.at[sl])
        plsc.subcore_barrier()

        half = NCH_E // 2

        def scan(cbase, climit):
            nsup = (climit - cbase + NS * DEPTH - 1) // (NS * DEPTH)

            @pl.loop(0, nsup)
            def _(j):
                base = cbase + t + NS * DEPTH * j
                for b in range(DEPTH):
                    c = base + NS * b

                    @pl.when(c < climit)
                    def _():
                        pltpu.async_copy(row_h.at[c], idx[b], sem[b])
                        pltpu.async_copy(
                            vv_h.at[pl.ds(c * CK, CK), pl.ds(F, WD)],
                            vbuf[b], sem[b])

                for b in range(DEPTH):
                    c = base + NS * b

                    @pl.when(c < climit)
                    def _():
                        pltpu.make_async_copy(
                            row_h.at[c], idx[b], sem[b]).wait()
                        pltpu.make_async_copy(
                            vv_h.at[pl.ds(c * CK, CK), pl.ds(F, WD)],
                            vbuf[b], sem[b]).wait()
                        pltpu.sync_copy(vbuf[b], acc.at[idx[b]], add=True)

        @pl.when(core == 0)
        def _():
            scan(0, half)

        @pl.when(core == 1)
        def _():
            scan(half, NCH_E)

        plsc.subcore_barrier()

        @pl.when(core == 0)
        def _():
            pltpu.sync_copy(acc.at[sl], d1_h.at[sl])

        @pl.when(core == 1)
        def _():
            pltpu.sync_copy(acc.at[sl], d2_h.at[sl])

    return k(row2d, vv, z8)


def _sc_batchsum(batch2d, nodevec, zb):
    scratch = [
        pltpu.VMEM_SHARED((B, 8), _f32),
        pltpu.VMEM((CKN,), jnp.int32),
        pltpu.VMEM((CKN, 8), _f32),
        pltpu.SemaphoreType.DMA,
    ]

    @functools.partial(
        pl.kernel,
        out_type=jax.ShapeDtypeStruct((NC, B, 8), _f32),
        mesh=_mesh(),
        scratch_types=scratch,
        compiler_params=_SC_LINEAR,
    )
    def k(batch_h, nv_h, zb_h, out_h, acc, idx, vbuf, sem):
        core = lax.axis_index("c")
        t = lax.axis_index("s")
        w = t * NC + core

        @pl.when(t == 0)
        def _():
            pltpu.sync_copy(zb_h, acc)

        plsc.subcore_barrier()

        nit = (NCH_N + NW - 1) // NW

        @pl.loop(0, nit)
        def _(j):
            c = w + j * NW

            @pl.when(c < NCH_N)
            def _():
                pltpu.async_copy(batch_h.at[c], idx, sem)
                pltpu.async_copy(nv_h.at[pl.ds(c * CKN, CKN)], vbuf, sem)
                pltpu.make_async_copy(batch_h.at[c], idx, sem).wait()
                pltpu.make_async_copy(
                    nv_h.at[pl.ds(c * CKN, CKN)], vbuf, sem).wait()
                pltpu.sync_copy(vbuf, acc.at[idx], add=True)

        plsc.subcore_barrier()

        @pl.when(t == 0)
        def _():
            pltpu.sync_copy(acc, out_h.at[core])

    return k(batch2d, nodevec, zb)


NB = 2000
EB = 4000


def _silu(v):
    return v * jax.nn.sigmoid(v)


def _full(shape):
    return pl.BlockSpec(shape, lambda *_: (0,) * len(shape))


def _tab(h, x, wa, wb, b1):
    zpad = jnp.zeros((h.shape[0], TW - 67), _f32)
    tr = jnp.concatenate([jnp.dot(h, wa) + b1, x, zpad], axis=1)
    tc = jnp.concatenate([jnp.dot(h, wb), x, zpad], axis=1)
    return tr, tc


def _tc_embed_prep(z2, pos, embP, W1a, W1b, b1):
    def body(z_r, pos_r, emb_r, wa_r, wb_r, b1_r, h_r, tr_r, tc_r):
        zb = z_r[...]
        iot = lax.broadcasted_iota(jnp.int32, (NB, 128), 1)
        oh = (iot == zb).astype(_f32)
        h0 = jnp.dot(oh, emb_r[...], preferred_element_type=_f32,
                     precision=lax.Precision.HIGHEST)
        h_r[...] = h0
        tr_r[...], tc_r[...] = _tab(h0, pos_r[...], wa_r[...], wb_r[...],
                                    b1_r[...])

    return pl.pallas_call(
        body,
        grid=(N // NB,),
        in_specs=[
            pl.BlockSpec((NB, 1), lambda i: (i, 0)),
            pl.BlockSpec((NB, 3), lambda i: (i, 0)),
            _full((128, F)), _full((F, H)), _full((F, H)), _full((1, H)),
        ],
        out_specs=[
            pl.BlockSpec((NB, F), lambda i: (i, 0)),
            pl.BlockSpec((NB, TW), lambda i: (i, 0)),
            pl.BlockSpec((NB, TW), lambda i: (i, 0)),
        ],
        out_shape=[
            jax.ShapeDtypeStruct((N, F), _f32),
            jax.ShapeDtypeStruct((N, TW), _f32),
            jax.ShapeDtypeStruct((N, TW), _f32),
        ],
    )(z2, pos, embP, W1a, W1b, b1)


def _node_update(h, x, a1, a2, d1, d2, wn1, bn1, wn2, bn2):
    u = jnp.concatenate([h, a1, a2], axis=1)
    t = _silu(jnp.dot(u, wn1, preferred_element_type=_f32) + bn1)
    h2 = h + jnp.dot(t, wn2, preferred_element_type=_f32) + bn2
    x2 = x + d1[:, :3] + d2[:, :3]
    return h2, x2


def _tc_update_prep(h, x, a1, a2, d1, d2, wn1, bn1, wn2, bn2, W1a, W1b, b1):
    def body(h_r, x_r, a1_r, a2_r, d1_r, d2_r, wn1_r, bn1_r, wn2_r, bn2_r,
             wa_r, wb_r, b1_r, h2_r, x2_r, tr_r, tc_r):
        h2, x2 = _node_update(h_r[...], x_r[...], a1_r[...], a2_r[...],
                              d1_r[...], d2_r[...],
                              wn1_r[...], bn1_r[...], wn2_r[...], bn2_r[...])
        h2_r[...] = h2
        x2_r[...] = x2
        tr_r[...], tc_r[...] = _tab(h2, x2, wa_r[...], wb_r[...], b1_r[...])

    return pl.pallas_call(
        body,
        grid=(N // NB,),
        in_specs=[
            pl.BlockSpec((NB, F), lambda i: (i, 0)),
            pl.BlockSpec((NB, 3), lambda i: (i, 0)),
            pl.BlockSpec((NB, WA), lambda i: (i, 0)),
            pl.BlockSpec((NB, WA), lambda i: (i, 0)),
            pl.BlockSpec((NB, WD), lambda i: (i, 0)),
            pl.BlockSpec((NB, WD), lambda i: (i, 0)),
            _full((2 * F, H)), _full((1, H)), _full((H, F)), _full((1, F)),
            _full((F, H)), _full((F, H)), _full((1, H)),
        ],
        out_specs=[
            pl.BlockSpec((NB, F), lambda i: (i, 0)),
            pl.BlockSpec((NB, 3), lambda i: (i, 0)),
            pl.BlockSpec((NB, TW), lambda i: (i, 0)),
            pl.BlockSpec((NB, TW), lambda i: (i, 0)),
        ],
        out_shape=[
            jax.ShapeDtypeStruct((N, F), _f32),
            jax.ShapeDtypeStruct((N, 3), _f32),
            jax.ShapeDtypeStruct((N, TW), _f32),
            jax.ShapeDtypeStruct((N, TW), _f32),
        ],
    )(h, x, a1, a2, d1, d2, wn1, bn1, wn2, bn2, W1a, W1b, b1)


def _tc_edge(gr, gc, W2, b2, Wc, bc, w1c):
    def body(gr_r, gc_r, w2_r, b2_r, wc_r, bc_r, w1c_r, vv_r):
        grv = gr_r[...]
        gcv = gc_r[...]
        rel = grv[:, 64:67] - gcv[:, 64:67]
        d2 = jnp.sum(rel * rel, axis=1, keepdims=True)
        d2b = d2.astype(jnp.bfloat16).astype(_f32)
        w1cb = w1c_r[...].astype(jnp.bfloat16).astype(_f32)
        pre = grv[:, :64] + gcv[:, :64] + d2b * w1cb
        t = _silu(pre)
        m = _silu(
            jnp.dot(t, w2_r[...], preferred_element_type=_f32) + b2_r[...])
        coef = jnp.tanh(
            jnp.dot(m, wc_r[...], preferred_element_type=_f32) + bc_r[...])
        trans = rel * coef
        vv_r[...] = jnp.concatenate(
            [m, trans, jnp.zeros((EB, TW - 67), _f32)], axis=1)

    return pl.pallas_call(
        body,
        grid=(E // EB,),
        in_specs=[
            pl.BlockSpec((EB, TW), lambda i: (i, 0)),
            pl.BlockSpec((EB, TW), lambda i: (i, 0)),
            _full((H, H)), _full((1, H)), _full((H, 1)), _full((1, 1)),
            _full((1, H)),
        ],
        out_specs=[pl.BlockSpec((EB, TW), lambda i: (i, 0))],
        out_shape=[jax.ShapeDtypeStruct((E, TW), _f32)],
    )(gr, gc, W2, b2, Wc, bc, w1c)[0]


def _tc_head(h, x, a1, a2, d1, d2, wn1, bn1, wn2, bn2, wc1, bc1, wc2, bc2):
    def body(h_r, x_r, a1_r, a2_r, d1_r, d2_r, wn1_r, bn1_r, wn2_r, bn2_r,
             wc1_r, bc1_r, wc2_r, bc2_r, nv_r):
        h2, x2 = _node_update(h_r[...], x_r[...], a1_r[...], a2_r[...],
                              d1_r[...], d2_r[...],
                              wn1_r[...], bn1_r[...], wn2_r[...], bn2_r[...])
        t = _silu(jnp.dot(h2, wc1_r[...], preferred_element_type=_f32)
                  + bc1_r[...])
        q = jnp.dot(t, wc2_r[...], preferred_element_type=_f32) + bc2_r[...]
        nv_r[...] = jnp.concatenate(
            [x2, q * x2, q, jnp.ones((NB, 1), _f32)], axis=1)

    return pl.pallas_call(
        body,
        grid=(N // NB,),
        in_specs=[
            pl.BlockSpec((NB, F), lambda i: (i, 0)),
            pl.BlockSpec((NB, 3), lambda i: (i, 0)),
            pl.BlockSpec((NB, WA), lambda i: (i, 0)),
            pl.BlockSpec((NB, WA), lambda i: (i, 0)),
            pl.BlockSpec((NB, WD), lambda i: (i, 0)),
            pl.BlockSpec((NB, WD), lambda i: (i, 0)),
            _full((2 * F, H)), _full((1, H)), _full((H, F)), _full((1, F)),
            _full((F, H)), _full((1, H)), _full((H, 1)), _full((1, 1)),
        ],
        out_specs=[pl.BlockSpec((NB, 8), lambda i: (i, 0))],
        out_shape=[jax.ShapeDtypeStruct((N, 8), _f32)],
    )(h, x, a1, a2, d1, d2, wn1, bn1, wn2, bn2, wc1, bc1, wc2, bc2)[0]


def _tc_mu(parts2):
    def body(p_r, mu_r):
        p = p_r[...]
        sTot = p[:B, :] + p[B:, :]
        cnt = jnp.maximum(sTot[:, 7:8], 1.0)
        mu_r[...] = sTot[:, 3:6] - sTot[:, :3] * (sTot[:, 6:7] / cnt)

    return pl.pallas_call(
        body,
        in_specs=[_full((2 * B, 8))],
        out_specs=pl.BlockSpec((B, 3), lambda: (0, 0)),
        out_shape=jax.ShapeDtypeStruct((B, 3), _f32),
    )(parts2)


def kernel(z, pos, edge_index, batch, params):
    row2d = edge_index[0].astype(jnp.int32).reshape(NCH_E, CK)
    col2d = edge_index[1].astype(jnp.int32).reshape(NCH_E, CK)
    batch2d = batch.astype(jnp.int32).reshape(NCH_N, CKN)
    z2 = z.astype(jnp.int32).reshape(N, 1)

    embP = jnp.pad(params["emb"], ((0, 128 - params["emb"].shape[0]), (0, 0)))
    zin32 = jnp.zeros((NPT, WA), _f32)
    zin8 = jnp.zeros((NPT, WD), _f32)
    zb = jnp.zeros((B, 8), _f32)

    def e1_split(p):
        W1 = p["e1"]["W"]
        return (W1[:F], W1[F:2 * F], p["e1"]["b"].reshape(1, H),
                W1[2 * F:2 * F + 1])

    layers = params["layers"]
    W1a, W1b, b1, w1c = e1_split(layers[0])
    h, tabR, tabC = _tc_embed_prep(z2, pos, embP, W1a, W1b, b1)
    x = pos

    for li, p in enumerate(layers):
        gr, gc = _sc_gather(tabR, tabC, row2d, col2d)
        vv = _tc_edge(
            gr, gc, p["e2"]["W"], p["e2"]["b"].reshape(1, H),
            p["c"]["W"], p["c"]["b"].reshape(1, 1), w1c)
        a1, a2 = _sc_scatter_m(row2d, vv, zin32)
        d1, d2 = _sc_scatter_t(row2d, vv, zin8)
        wn1 = p["n1"]["W"]
        bn1 = p["n1"]["b"].reshape(1, H)
        wn2 = p["n2"]["W"]
        bn2 = p["n2"]["b"].reshape(1, F)
        if li + 1 < len(layers):
            W1a, W1b, b1, w1c = e1_split(layers[li + 1])
            h, x, tabR, tabC = _tc_update_prep(
                h, x, a1, a2, d1, d2, wn1, bn1, wn2, bn2, W1a, W1b, b1)
        else:
            ch = params["charge"]
            nodevec = _tc_head(
                h, x, a1, a2, d1, d2, wn1, bn1, wn2, bn2,
                ch["l1"]["W"], ch["l1"]["b"].reshape(1, H),
                ch["l2"]["W"], ch["l2"]["b"].reshape(1, 1))

    parts = _sc_batchsum(batch2d, nodevec, zb)
    return _tc_mu(parts.reshape(2 * B, 8))


# speedup vs baseline: 1.6136x; 1.6136x over previous
import functools

import jax
import jax.numpy as jnp
from jax import lax
from jax.experimental import pallas as pl
from jax.experimental.pallas import tpu as pltpu
from jax.experimental.pallas import tpu_sc as plsc

N = 50000
E = 800000
F = 64
H = 64
B = 512

NC = 2
NS = 16
NW = NC * NS

CK = 128
NCH_E = E // CK
CKN = 80
NCH_N = N // CKN
NPT = N // NS
WA = 32

_f32 = jnp.float32

_SC_LINEAR = pltpu.CompilerParams(use_tc_tiling_on_sc=False)


def _mesh():
    return plsc.VectorSubcoreMesh(core_axis_name="c", subcore_axis_name="s")


def _sc_gather(tabR, tabC, row2d, col2d):
    DEPTH = 2
    scratch = []
    for _ in range(DEPTH):
        scratch += [
            pltpu.VMEM((CK,), jnp.int32),
            pltpu.VMEM((CK,), jnp.int32),
            pltpu.VMEM((CK, 128), _f32),
            pltpu.VMEM((CK, 128), _f32),
            pltpu.SemaphoreType.DMA,
            pltpu.SemaphoreType.DMA,
        ]

    @functools.partial(
        pl.kernel,
        out_type=(
            jax.ShapeDtypeStruct((E, 128), _f32),
            jax.ShapeDtypeStruct((E, 128), _f32),
        ),
        mesh=_mesh(),
        scratch_types=scratch,
    )
    def k(tabR_h, tabC_h, row_h, col_h, gr_h, gc_h, *s):
        w = lax.axis_index("s") * NC + lax.axis_index("c")
        idxR = s[0::6]
        idxC = s[1::6]
        bufR = s[2::6]
        bufC = s[3::6]
        semI = s[4::6]
        semG = s[5::6]
        nsup = (NCH_E + NW * DEPTH - 1) // (NW * DEPTH)

        @pl.loop(0, nsup)
        def _(j):
            base = w + j * (NW * DEPTH)
            for b in range(DEPTH):
                c = base + b * NW

                @pl.when(c < NCH_E)
                def _():
                    pltpu.async_copy(row_h.at[c], idxR[b], semI[b])
                    pltpu.async_copy(col_h.at[c], idxC[b], semI[b])

            for b in range(DEPTH):
                c = base + b * NW

                @pl.when(c < NCH_E)
                def _():
                    pltpu.make_async_copy(row_h.at[c], idxR[b], semI[b]).wait()
                    pltpu.make_async_copy(col_h.at[c], idxC[b], semI[b]).wait()
                    pltpu.async_copy(tabR_h.at[idxR[b]], bufR[b], semG[b])
                    pltpu.async_copy(tabC_h.at[idxC[b]], bufC[b], semG[b])

            for b in range(DEPTH):
                c = base + b * NW

                @pl.when(c < NCH_E)
                def _():
                    pltpu.make_async_copy(
                        tabR_h.at[idxR[b]], bufR[b], semG[b]).wait()
                    pltpu.make_async_copy(
                        tabC_h.at[idxC[b]], bufC[b], semG[b]).wait()
                    pltpu.sync_copy(bufR[b], gr_h.at[pl.ds(c * CK, CK)])
                    pltpu.sync_copy(bufC[b], gc_h.at[pl.ds(c * CK, CK)])

    return k(tabR, tabC, row2d, col2d)


def _sc_scatter(row2d, vv, zin):
    DEPTH = 4
    scratch = [pltpu.VMEM_SHARED((N, WA), _f32)]
    for _ in range(DEPTH):
        scratch += [
            pltpu.VMEM((CK,), jnp.int32),
            pltpu.VMEM((CK, WA), _f32),
            pltpu.SemaphoreType.DMA,
        ]

    @functools.partial(
        pl.kernel,
        out_type=(
            jax.ShapeDtypeStruct((N, WA), _f32),
            jax.ShapeDtypeStruct((N, WA), _f32),
            jax.ShapeDtypeStruct((N, WA), _f32),
            jax.ShapeDtypeStruct((N, WA), _f32),
        ),
        mesh=_mesh(),
        scratch_types=scratch,
        compiler_params=_SC_LINEAR,
    )
    def k(row_h, vv_h, zin_h, a1_h, a2_h, d1_h, d2_h, *s):
        core = lax.axis_index("c")
        t = lax.axis_index("s")
        acc = s[0]
        idx = s[1::3]
        vbuf = s[2::3]
        sem = s[3::3]
        sl = pl.ds(t * NPT, NPT)

        def zero_acc():
            pltpu.sync_copy(zin_h, acc.at[sl])
            plsc.subcore_barrier()

        def scan(coff, cbase, climit, stride):
            nsup = (climit - cbase + stride * DEPTH - 1) // (stride * DEPTH)

            @pl.loop(0, nsup)
            def _(j):
                base = cbase + t + stride * DEPTH * j
                for b in range(DEPTH):
                    c = base + stride * b

                    @pl.when(c < climit)
                    def _():
                        pltpu.async_copy(row_h.at[c], idx[b], sem[b])
                        pltpu.async_copy(
                            vv_h.at[pl.ds(c * CK, CK), pl.ds(coff, WA)],
                            vbuf[b], sem[b])

                for b in range(DEPTH):
                    c = base + stride * b

                    @pl.when(c < climit)
                    def _():
                        pltpu.make_async_copy(
                            row_h.at[c], idx[b], sem[b]).wait()
                        pltpu.make_async_copy(
                            vv_h.at[pl.ds(c * CK, CK), pl.ds(coff, WA)],
                            vbuf[b], sem[b]).wait()
                        pltpu.sync_copy(vbuf[b], acc.at[idx[b]], add=True)

        zero_acc()

        @pl.when(core == 0)
        def _():
            scan(0, 0, NCH_E, NS)

        @pl.when(core == 1)
        def _():
            scan(WA, 0, NCH_E, NS)

        plsc.subcore_barrier()

        @pl.when(core == 0)
        def _():
            pltpu.sync_copy(acc.at[sl], a1_h.at[sl])

        @pl.when(core == 1)
        def _():
            pltpu.sync_copy(acc.at[sl], a2_h.at[sl])

        plsc.subcore_barrier()

        zero_acc()
        half = NCH_E // 2

        @pl.when(core == 0)
        def _():
            scan(2 * WA, 0, half, NS)

        @pl.when(core == 1)
        def _():
            scan(2 * WA, half, NCH_E, NS)

        plsc.subcore_barrier()

        @pl.when(core == 0)
        def _():
            pltpu.sync_copy(acc.at[sl], d1_h.at[sl])

        @pl.when(core == 1)
        def _():
            pltpu.sync_copy(acc.at[sl], d2_h.at[sl])

    return k(row2d, vv, zin)


def _sc_batchsum(batch2d, nodevec, zb):
    scratch = [
        pltpu.VMEM_SHARED((B, 8), _f32),
        pltpu.VMEM((CKN,), jnp.int32),
        pltpu.VMEM((CKN, 8), _f32),
        pltpu.SemaphoreType.DMA,
    ]

    @functools.partial(
        pl.kernel,
        out_type=jax.ShapeDtypeStruct((NC, B, 8), _f32),
        mesh=_mesh(),
        scratch_types=scratch,
        compiler_params=_SC_LINEAR,
    )
    def k(batch_h, nv_h, zb_h, out_h, acc, idx, vbuf, sem):
        core = lax.axis_index("c")
        t = lax.axis_index("s")
        w = t * NC + core

        @pl.when(t == 0)
        def _():
            pltpu.sync_copy(zb_h, acc)

        plsc.subcore_barrier()

        nit = (NCH_N + NW - 1) // NW

        @pl.loop(0, nit)
        def _(j):
            c = w + j * NW

            @pl.when(c < NCH_N)
            def _():
                pltpu.async_copy(batch_h.at[c], idx, sem)
                pltpu.async_copy(nv_h.at[pl.ds(c * CKN, CKN)], vbuf, sem)
                pltpu.make_async_copy(batch_h.at[c], idx, sem).wait()
                pltpu.make_async_copy(
                    nv_h.at[pl.ds(c * CKN, CKN)], vbuf, sem).wait()
                pltpu.sync_copy(vbuf, acc.at[idx], add=True)

        plsc.subcore_barrier()

        @pl.when(t == 0)
        def _():
            pltpu.sync_copy(acc, out_h.at[core])

    return k(batch2d, nodevec, zb)


NB = 2000
EB = 4000


def _silu(v):
    return v * jax.nn.sigmoid(v)


def _full(shape):
    return pl.BlockSpec(shape, lambda *_: (0,) * len(shape))


def _tab(h, x, wa, wb, b1):
    zpad = jnp.zeros((h.shape[0], 61), _f32)
    tr = jnp.concatenate([jnp.dot(h, wa) + b1, x, zpad], axis=1)
    tc = jnp.concatenate([jnp.dot(h, wb), x, zpad], axis=1)
    return tr, tc


def _tc_embed_prep(z2, pos, embP, W1a, W1b, b1):
    def body(z_r, pos_r, emb_r, wa_r, wb_r, b1_r, h_r, tr_r, tc_r):
        zb = z_r[...]
        iot = lax.broadcasted_iota(jnp.int32, (NB, 128), 1)
        oh = (iot == zb).astype(_f32)
        h0 = jnp.dot(oh, emb_r[...], preferred_element_type=_f32,
                     precision=lax.Precision.HIGHEST)
        h_r[...] = h0
        tr_r[...], tc_r[...] = _tab(h0, pos_r[...], wa_r[...], wb_r[...],
                                    b1_r[...])

    return pl.pallas_call(
        body,
        grid=(N // NB,),
        in_specs=[
            pl.BlockSpec((NB, 1), lambda i: (i, 0)),
            pl.BlockSpec((NB, 3), lambda i: (i, 0)),
            _full((128, F)), _full((F, H)), _full((F, H)), _full((1, H)),
        ],
        out_specs=[
            pl.BlockSpec((NB, F), lambda i: (i, 0)),
            pl.BlockSpec((NB, 128), lambda i: (i, 0)),
            pl.BlockSpec((NB, 128), lambda i: (i, 0)),
        ],
        out_shape=[
            jax.ShapeDtypeStruct((N, F), _f32),
            jax.ShapeDtypeStruct((N, 128), _f32),
            jax.ShapeDtypeStruct((N, 128), _f32),
        ],
    )(z2, pos, embP, W1a, W1b, b1)


def _node_update(h, x, a1, a2, d1, d2, wn1, bn1, wn2, bn2):
    u = jnp.concatenate([h, a1, a2], axis=1)
    t = _silu(jnp.dot(u, wn1, preferred_element_type=_f32) + bn1)
    h2 = h + jnp.dot(t, wn2, preferred_element_type=_f32) + bn2
    x2 = x + d1[:, :3] + d2[:, :3]
    return h2, x2


def _tc_update_prep(h, x, a1, a2, d1, d2, wn1, bn1, wn2, bn2, W1a, W1b, b1):
    def body(h_r, x_r, a1_r, a2_r, d1_r, d2_r, wn1_r, bn1_r, wn2_r, bn2_r,
             wa_r, wb_r, b1_r, h2_r, x2_r, tr_r, tc_r):
        h2, x2 = _node_update(h_r[...], x_r[...], a1_r[...], a2_r[...],
                              d1_r[...], d2_r[...],
                              wn1_r[...], bn1_r[...], wn2_r[...], bn2_r[...])
        h2_r[...] = h2
        x2_r[...] = x2
        tr_r[...], tc_r[...] = _tab(h2, x2, wa_r[...], wb_r[...], b1_r[...])

    return pl.pallas_call(
        body,
        grid=(N // NB,),
        in_specs=[
            pl.BlockSpec((NB, F), lambda i: (i, 0)),
            pl.BlockSpec((NB, 3), lambda i: (i, 0)),
            pl.BlockSpec((NB, WA), lambda i: (i, 0)),
            pl.BlockSpec((NB, WA), lambda i: (i, 0)),
            pl.BlockSpec((NB, WA), lambda i: (i, 0)),
            pl.BlockSpec((NB, WA), lambda i: (i, 0)),
            _full((2 * F, H)), _full((1, H)), _full((H, F)), _full((1, F)),
            _full((F, H)), _full((F, H)), _full((1, H)),
        ],
        out_specs=[
            pl.BlockSpec((NB, F), lambda i: (i, 0)),
            pl.BlockSpec((NB, 3), lambda i: (i, 0)),
            pl.BlockSpec((NB, 128), lambda i: (i, 0)),
            pl.BlockSpec((NB, 128), lambda i: (i, 0)),
        ],
        out_shape=[
            jax.ShapeDtypeStruct((N, F), _f32),
            jax.ShapeDtypeStruct((N, 3), _f32),
            jax.ShapeDtypeStruct((N, 128), _f32),
            jax.ShapeDtypeStruct((N, 128), _f32),
        ],
    )(h, x, a1, a2, d1, d2, wn1, bn1, wn2, bn2, W1a, W1b, b1)


def _tc_edge(gr, gc, W2, b2, Wc, bc, w1c):
    def body(gr_r, gc_r, w2_r, b2_r, wc_r, bc_r, w1c_r, vv_r):
        grv = gr_r[...]
        gcv = gc_r[...]
        rel = grv[:, 64:67] - gcv[:, 64:67]
        d2 = jnp.sum(rel * rel, axis=1, keepdims=True)
        d2b = d2.astype(jnp.bfloat16).astype(_f32)
        w1cb = w1c_r[...].astype(jnp.bfloat16).astype(_f32)
        pre = grv[:, :64] + gcv[:, :64] + d2b * w1cb
        t = _silu(pre)
        m = _silu(
            jnp.dot(t, w2_r[...], preferred_element_type=_f32) + b2_r[...])
        coef = jnp.tanh(
            jnp.dot(m, wc_r[...], preferred_element_type=_f32) + bc_r[...])
        trans = rel * coef
        vv_r[...] = jnp.concatenate(
            [m, trans, jnp.zeros((EB, 61), _f32)], axis=1)

    return pl.pallas_call(
        body,
        grid=(E // EB,),
        in_specs=[
            pl.BlockSpec((EB, 128), lambda i: (i, 0)),
            pl.BlockSpec((EB, 128), lambda i: (i, 0)),
            _full((H, H)), _full((1, H)), _full((H, 1)), _full((1, 1)),
            _full((1, H)),
        ],
        out_specs=[pl.BlockSpec((EB, 128), lambda i: (i, 0))],
        out_shape=[jax.ShapeDtypeStruct((E, 128), _f32)],
    )(gr, gc, W2, b2, Wc, bc, w1c)[0]


def _tc_head(h, x, a1, a2, d1, d2, wn1, bn1, wn2, bn2, wc1, bc1, wc2, bc2):
    def body(h_r, x_r, a1_r, a2_r, d1_r, d2_r, wn1_r, bn1_r, wn2_r, bn2_r,
             wc1_r, bc1_r, wc2_r, bc2_r, nv_r):
        h2, x2 = _node_update(h_r[...], x_r[...], a1_r[...], a2_r[...],
                              d1_r[...], d2_r[...],
                              wn1_r[...], bn1_r[...], wn2_r[...], bn2_r[...])
        t = _silu(jnp.dot(h2, wc1_r[...], preferred_element_type=_f32)
                  + bc1_r[...])
        q = jnp.dot(t, wc2_r[...], preferred_element_type=_f32) + bc2_r[...]
        nv_r[...] = jnp.concatenate(
            [x2, q * x2, q, jnp.ones((NB, 1), _f32)], axis=1)

    return pl.pallas_call(
        body,
        grid=(N // NB,),
        in_specs=[
            pl.BlockSpec((NB, F), lambda i: (i, 0)),
            pl.BlockSpec((NB, 3), lambda i: (i, 0)),
            pl.BlockSpec((NB, WA), lambda i: (i, 0)),
            pl.BlockSpec((NB, WA), lambda i: (i, 0)),
            pl.BlockSpec((NB, WA), lambda i: (i, 0)),
            pl.BlockSpec((NB, WA), lambda i: (i, 0)),
            _full((2 * F, H)), _full((1, H)), _full((H, F)), _full((1, F)),
            _full((F, H)), _full((1, H)), _full((H, 1)), _full((1, 1)),
        ],
        out_specs=[pl.BlockSpec((NB, 8), lambda i: (i, 0))],
        out_shape=[jax.ShapeDtypeStruct((N, 8), _f32)],
    )(h, x, a1, a2, d1, d2, wn1, bn1, wn2, bn2, wc1, bc1, wc2, bc2)[0]


def _tc_mu(parts2):
    def body(p_r, mu_r):
        p = p_r[...]
        sTot = p[:B, :] + p[B:, :]
        cnt = jnp.maximum(sTot[:, 7:8], 1.0)
        mu_r[...] = sTot[:, 3:6] - sTot[:, :3] * (sTot[:, 6:7] / cnt)

    return pl.pallas_call(
        body,
        in_specs=[_full((2 * B, 8))],
        out_specs=pl.BlockSpec((B, 3), lambda: (0, 0)),
        out_shape=jax.ShapeDtypeStruct((B, 3), _f32),
    )(parts2)


def kernel(z, pos, edge_index, batch, params):
    row2d = edge_index[0].astype(jnp.int32).reshape(NCH_E, CK)
    col2d = edge_index[1].astype(jnp.int32).reshape(NCH_E, CK)
    batch2d = batch.astype(jnp.int32).reshape(NCH_N, CKN)
    z2 = z.astype(jnp.int32).reshape(N, 1)

    embP = jnp.pad(params["emb"], ((0, 128 - params["emb"].shape[0]), (0, 0)))
    zin = jnp.zeros((NPT, WA), _f32)
    zb = jnp.zeros((B, 8), _f32)

    def e1_split(p):
        W1 = p["e1"]["W"]
        return (W1[:F], W1[F:2 * F], p["e1"]["b"].reshape(1, H),
                W1[2 * F:2 * F + 1])

    layers = params["layers"]
    W1a, W1b, b1, w1c = e1_split(layers[0])
    h, tabR, tabC = _tc_embed_prep(z2, pos, embP, W1a, W1b, b1)
    x = pos

    for li, p in enumerate(layers):
        gr, gc = _sc_gather(tabR, tabC, row2d, col2d)
        vv = _tc_edge(
            gr, gc, p["e2"]["W"], p["e2"]["b"].reshape(1, H),
            p["c"]["W"], p["c"]["b"].reshape(1, 1), w1c)
        a1, a2, d1, d2 = _sc_scatter(row2d, vv, zin)
        wn1 = p["n1"]["W"]
        bn1 = p["n1"]["b"].reshape(1, H)
        wn2 = p["n2"]["W"]
        bn2 = p["n2"]["b"].reshape(1, F)
        if li + 1 < len(layers):
            W1a, W1b, b1, w1c = e1_split(layers[li + 1])
            h, x, tabR, tabC = _tc_update_prep(
                h, x, a1, a2, d1, d2, wn1, bn1, wn2, bn2, W1a, W1b, b1)
        else:
            ch = params["charge"]
            nodevec = _tc_head(
                h, x, a1, a2, d1, d2, wn1, bn1, wn2, bn2,
                ch["l1"]["W"], ch["l1"]["b"].reshape(1, H),
                ch["l2"]["W"], ch["l2"]["b"].reshape(1, 1))

    parts = _sc_batchsum(batch2d, nodevec, zb)
    return _tc_mu(parts.reshape(2 * B, 8))


# speedup vs baseline: 1.8743x; 1.1615x over previous
import functools

import jax
import jax.numpy as jnp
from jax import lax
from jax.experimental import pallas as pl
from jax.experimental.pallas import tpu as pltpu
from jax.experimental.pallas import tpu_sc as plsc

N = 50000
E = 800000
F = 64
H = 64
B = 512

NC = 2
NS = 16
NW = NC * NS

CK = 128
NCH_E = E // CK
CKN = 80
NCH_N = N // CKN
NPT = N // NS
WA = 32

_f32 = jnp.float32

_SC_LINEAR = pltpu.CompilerParams(use_tc_tiling_on_sc=False)


def _mesh():
    return plsc.VectorSubcoreMesh(core_axis_name="c", subcore_axis_name="s")


def _sc_gather(tabR, tabC, row2d, col2d):
    DEPTH = 2
    nch = row2d.shape[0]
    ne = nch * CK
    scratch = []
    for _ in range(DEPTH):
        scratch += [
            pltpu.VMEM((CK,), jnp.int32),
            pltpu.VMEM((CK,), jnp.int32),
            pltpu.VMEM((CK, 128), _f32),
            pltpu.VMEM((CK, 128), _f32),
            pltpu.SemaphoreType.DMA,
            pltpu.SemaphoreType.DMA,
        ]

    @functools.partial(
        pl.kernel,
        out_type=(
            jax.ShapeDtypeStruct((ne, 128), _f32),
            jax.ShapeDtypeStruct((ne, 128), _f32),
        ),
        mesh=_mesh(),
        scratch_types=scratch,
    )
    def k(tabR_h, tabC_h, row_h, col_h, gr_h, gc_h, *s):
        w = lax.axis_index("s") * NC + lax.axis_index("c")
        idxR = s[0::6]
        idxC = s[1::6]
        bufR = s[2::6]
        bufC = s[3::6]
        semI = s[4::6]
        semG = s[5::6]
        nsup = (nch + NW * DEPTH - 1) // (NW * DEPTH)

        @pl.loop(0, nsup)
        def _(j):
            base = w + j * (NW * DEPTH)
            for b in range(DEPTH):
                c = base + b * NW

                @pl.when(c < nch)
                def _():
                    pltpu.async_copy(row_h.at[c], idxR[b], semI[b])
                    pltpu.async_copy(col_h.at[c], idxC[b], semI[b])

            for b in range(DEPTH):
                c = base + b * NW

                @pl.when(c < nch)
                def _():
                    pltpu.make_async_copy(row_h.at[c], idxR[b], semI[b]).wait()
                    pltpu.make_async_copy(col_h.at[c], idxC[b], semI[b]).wait()
                    pltpu.async_copy(tabR_h.at[idxR[b]], bufR[b], semG[b])
                    pltpu.async_copy(tabC_h.at[idxC[b]], bufC[b], semG[b])

            for b in range(DEPTH):
                c = base + b * NW

                @pl.when(c < nch)
                def _():
                    pltpu.make_async_copy(
                        tabR_h.at[idxR[b]], bufR[b], semG[b]).wait()
                    pltpu.make_async_copy(
                        tabC_h.at[idxC[b]], bufC[b], semG[b]).wait()
                    pltpu.sync_copy(bufR[b], gr_h.at[pl.ds(c * CK, CK)])
                    pltpu.sync_copy(bufC[b], gc_h.at[pl.ds(c * CK, CK)])

    return k(tabR, tabC, row2d, col2d)


def _sc_scatter(row2d, vv, aI1, aI2, dI1, dI2):
    DEPTH = 4
    nch = row2d.shape[0]
    scratch = [pltpu.VMEM_SHARED((N, WA), _f32)]
    for _ in range(DEPTH):
        scratch += [
            pltpu.VMEM((CK,), jnp.int32),
            pltpu.VMEM((CK, WA), _f32),
            pltpu.SemaphoreType.DMA,
        ]

    @functools.partial(
        pl.kernel,
        out_type=(
            jax.ShapeDtypeStruct((N, WA), _f32),
            jax.ShapeDtypeStruct((N, WA), _f32),
            jax.ShapeDtypeStruct((N, WA), _f32),
            jax.ShapeDtypeStruct((N, WA), _f32),
        ),
        mesh=_mesh(),
        scratch_types=scratch,
        compiler_params=_SC_LINEAR,
    )
    def k(row_h, vv_h, aI1_h, aI2_h, dI1_h, dI2_h,
          a1_h, a2_h, d1_h, d2_h, *s):
        core = lax.axis_index("c")
        t = lax.axis_index("s")
        acc = s[0]
        idx = s[1::3]
        vbuf = s[2::3]
        sem = s[3::3]
        sl = pl.ds(t * NPT, NPT)

        def init_acc(i1_h, i2_h):
            @pl.when(core == 0)
            def _():
                pltpu.sync_copy(i1_h.at[sl], acc.at[sl])

            @pl.when(core == 1)
            def _():
                pltpu.sync_copy(i2_h.at[sl], acc.at[sl])

            plsc.subcore_barrier()

        def scan(coff, cbase, climit, stride):
            nsup = (climit - cbase + stride * DEPTH - 1) // (stride * DEPTH)

            @pl.loop(0, nsup)
            def _(j):
                base = cbase + t + stride * DEPTH * j
                for b in range(DEPTH):
                    c = base + stride * b

                    @pl.when(c < climit)
                    def _():
                        pltpu.async_copy(row_h.at[c], idx[b], sem[b])
                        pltpu.async_copy(
                            vv_h.at[pl.ds(c * CK, CK), pl.ds(coff, WA)],
                            vbuf[b], sem[b])

                for b in range(DEPTH):
                    c = base + stride * b

                    @pl.when(c < climit)
                    def _():
                        pltpu.make_async_copy(
                            row_h.at[c], idx[b], sem[b]).wait()
                        pltpu.make_async_copy(
                            vv_h.at[pl.ds(c * CK, CK), pl.ds(coff, WA)],
                            vbuf[b], sem[b]).wait()
                        pltpu.sync_copy(vbuf[b], acc.at[idx[b]], add=True)

        init_acc(aI1_h, aI2_h)

        @pl.when(core == 0)
        def _():
            scan(0, 0, nch, NS)

        @pl.when(core == 1)
        def _():
            scan(WA, 0, nch, NS)

        plsc.subcore_barrier()

        @pl.when(core == 0)
        def _():
            pltpu.sync_copy(acc.at[sl], a1_h.at[sl])

        @pl.when(core == 1)
        def _():
            pltpu.sync_copy(acc.at[sl], a2_h.at[sl])

        plsc.subcore_barrier()

        init_acc(dI1_h, dI2_h)
        half = nch // 2

        @pl.when(core == 0)
        def _():
            scan(2 * WA, 0, half, NS)

        @pl.when(core == 1)
        def _():
            scan(2 * WA, half, nch, NS)

        plsc.subcore_barrier()

        @pl.when(core == 0)
        def _():
            pltpu.sync_copy(acc.at[sl], d1_h.at[sl])

        @pl.when(core == 1)
        def _():
            pltpu.sync_copy(acc.at[sl], d2_h.at[sl])

    return k(row2d, vv, aI1, aI2, dI1, dI2)


def _sc_batchsum(batch2d, nodevec, zb):
    scratch = [
        pltpu.VMEM_SHARED((B, 8), _f32),
        pltpu.VMEM((CKN,), jnp.int32),
        pltpu.VMEM((CKN, 8), _f32),
        pltpu.SemaphoreType.DMA,
    ]

    @functools.partial(
        pl.kernel,
        out_type=jax.ShapeDtypeStruct((NC, B, 8), _f32),
        mesh=_mesh(),
        scratch_types=scratch,
        compiler_params=_SC_LINEAR,
    )
    def k(batch_h, nv_h, zb_h, out_h, acc, idx, vbuf, sem):
        core = lax.axis_index("c")
        t = lax.axis_index("s")
        w = t * NC + core

        @pl.when(t == 0)
        def _():
            pltpu.sync_copy(zb_h, acc)

        plsc.subcore_barrier()

        nit = (NCH_N + NW - 1) // NW

        @pl.loop(0, nit)
        def _(j):
            c = w + j * NW

            @pl.when(c < NCH_N)
            def _():
                pltpu.async_copy(batch_h.at[c], idx, sem)
                pltpu.async_copy(nv_h.at[pl.ds(c * CKN, CKN)], vbuf, sem)
                pltpu.make_async_copy(batch_h.at[c], idx, sem).wait()
                pltpu.make_async_copy(
                    nv_h.at[pl.ds(c * CKN, CKN)], vbuf, sem).wait()
                pltpu.sync_copy(vbuf, acc.at[idx], add=True)

        plsc.subcore_barrier()

        @pl.when(t == 0)
        def _():
            pltpu.sync_copy(acc, out_h.at[core])

    return k(batch2d, nodevec, zb)


NB = 2000
EB = 4000


def _silu(v):
    return v * jax.nn.sigmoid(v)


def _full(shape):
    return pl.BlockSpec(shape, lambda *_: (0,) * len(shape))


def _tab(h, x, wa, wb, b1):
    zpad = jnp.zeros((h.shape[0], 61), _f32)
    tr = jnp.concatenate([jnp.dot(h, wa) + b1, x, zpad], axis=1)
    tc = jnp.concatenate([jnp.dot(h, wb), x, zpad], axis=1)
    return tr, tc


def _tc_embed_prep(z2, pos, embP, W1a, W1b, b1):
    def body(z_r, pos_r, emb_r, wa_r, wb_r, b1_r, h_r, tr_r, tc_r):
        zb = z_r[...]
        iot = lax.broadcasted_iota(jnp.int32, (NB, 128), 1)
        oh = (iot == zb).astype(_f32)
        h0 = jnp.dot(oh, emb_r[...], preferred_element_type=_f32,
                     precision=lax.Precision.HIGHEST)
        h_r[...] = h0
        tr_r[...], tc_r[...] = _tab(h0, pos_r[...], wa_r[...], wb_r[...],
                                    b1_r[...])

    return pl.pallas_call(
        body,
        grid=(N // NB,),
        in_specs=[
            pl.BlockSpec((NB, 1), lambda i: (i, 0)),
            pl.BlockSpec((NB, 3), lambda i: (i, 0)),
            _full((128, F)), _full((F, H)), _full((F, H)), _full((1, H)),
        ],
        out_specs=[
            pl.BlockSpec((NB, F), lambda i: (i, 0)),
            pl.BlockSpec((NB, 128), lambda i: (i, 0)),
            pl.BlockSpec((NB, 128), lambda i: (i, 0)),
        ],
        out_shape=[
            jax.ShapeDtypeStruct((N, F), _f32),
            jax.ShapeDtypeStruct((N, 128), _f32),
            jax.ShapeDtypeStruct((N, 128), _f32),
        ],
    )(z2, pos, embP, W1a, W1b, b1)


def _node_update(h, x, a1, a2, d1, d2, wn1, bn1, wn2, bn2):
    u = jnp.concatenate([h, a1, a2], axis=1)
    t = _silu(jnp.dot(u, wn1, preferred_element_type=_f32) + bn1)
    h2 = h + jnp.dot(t, wn2, preferred_element_type=_f32) + bn2
    x2 = x + d1[:, :3] + d2[:, :3]
    return h2, x2


def _tc_update_prep(h, x, a1, a2, d1, d2, wn1, bn1, wn2, bn2, W1a, W1b, b1):
    def body(h_r, x_r, a1_r, a2_r, d1_r, d2_r, wn1_r, bn1_r, wn2_r, bn2_r,
             wa_r, wb_r, b1_r, h2_r, x2_r, tr_r, tc_r):
        h2, x2 = _node_update(h_r[...], x_r[...], a1_r[...], a2_r[...],
                              d1_r[...], d2_r[...],
                              wn1_r[...], bn1_r[...], wn2_r[...], bn2_r[...])
        h2_r[...] = h2
        x2_r[...] = x2
        tr_r[...], tc_r[...] = _tab(h2, x2, wa_r[...], wb_r[...], b1_r[...])

    return pl.pallas_call(
        body,
        grid=(N // NB,),
        in_specs=[
            pl.BlockSpec((NB, F), lambda i: (i, 0)),
            pl.BlockSpec((NB, 3), lambda i: (i, 0)),
            pl.BlockSpec((NB, WA), lambda i: (i, 0)),
            pl.BlockSpec((NB, WA), lambda i: (i, 0)),
            pl.BlockSpec((NB, WA), lambda i: (i, 0)),
            pl.BlockSpec((NB, WA), lambda i: (i, 0)),
            _full((2 * F, H)), _full((1, H)), _full((H, F)), _full((1, F)),
            _full((F, H)), _full((F, H)), _full((1, H)),
        ],
        out_specs=[
            pl.BlockSpec((NB, F), lambda i: (i, 0)),
            pl.BlockSpec((NB, 3), lambda i: (i, 0)),
            pl.BlockSpec((NB, 128), lambda i: (i, 0)),
            pl.BlockSpec((NB, 128), lambda i: (i, 0)),
        ],
        out_shape=[
            jax.ShapeDtypeStruct((N, F), _f32),
            jax.ShapeDtypeStruct((N, 3), _f32),
            jax.ShapeDtypeStruct((N, 128), _f32),
            jax.ShapeDtypeStruct((N, 128), _f32),
        ],
    )(h, x, a1, a2, d1, d2, wn1, bn1, wn2, bn2, W1a, W1b, b1)


def _tc_edge(gr, gc, W2, b2, Wc, bc, w1c):
    ne = gr.shape[0]

    def body(gr_r, gc_r, w2_r, b2_r, wc_r, bc_r, w1c_r, vv_r):
        grv = gr_r[...]
        gcv = gc_r[...]
        rel = grv[:, 64:67] - gcv[:, 64:67]
        d2 = jnp.sum(rel * rel, axis=1, keepdims=True)
        d2b = d2.astype(jnp.bfloat16).astype(_f32)
        w1cb = w1c_r[...].astype(jnp.bfloat16).astype(_f32)
        pre = grv[:, :64] + gcv[:, :64] + d2b * w1cb
        t = _silu(pre)
        m = _silu(
            jnp.dot(t, w2_r[...], preferred_element_type=_f32) + b2_r[...])
        coef = jnp.tanh(
            jnp.dot(m, wc_r[...], preferred_element_type=_f32) + bc_r[...])
        trans = rel * coef
        vv_r[...] = jnp.concatenate(
            [m, trans, jnp.zeros((EB, 61), _f32)], axis=1)

    return pl.pallas_call(
        body,
        grid=(ne // EB,),
        in_specs=[
            pl.BlockSpec((EB, 128), lambda i: (i, 0)),
            pl.BlockSpec((EB, 128), lambda i: (i, 0)),
            _full((H, H)), _full((1, H)), _full((H, 1)), _full((1, 1)),
            _full((1, H)),
        ],
        out_specs=[pl.BlockSpec((EB, 128), lambda i: (i, 0))],
        out_shape=[jax.ShapeDtypeStruct((ne, 128), _f32)],
    )(gr, gc, W2, b2, Wc, bc, w1c)[0]


def _tc_head(h, x, a1, a2, d1, d2, wn1, bn1, wn2, bn2, wc1, bc1, wc2, bc2):
    def body(h_r, x_r, a1_r, a2_r, d1_r, d2_r, wn1_r, bn1_r, wn2_r, bn2_r,
             wc1_r, bc1_r, wc2_r, bc2_r, nv_r):
        h2, x2 = _node_update(h_r[...], x_r[...], a1_r[...], a2_r[...],
                              d1_r[...], d2_r[...],
                              wn1_r[...], bn1_r[...], wn2_r[...], bn2_r[...])
        t = _silu(jnp.dot(h2, wc1_r[...], preferred_element_type=_f32)
                  + bc1_r[...])
        q = jnp.dot(t, wc2_r[...], preferred_element_type=_f32) + bc2_r[...]
        nv_r[...] = jnp.concatenate(
            [x2, q * x2, q, jnp.ones((NB, 1), _f32)], axis=1)

    return pl.pallas_call(
        body,
        grid=(N // NB,),
        in_specs=[
            pl.BlockSpec((NB, F), lambda i: (i, 0)),
            pl.BlockSpec((NB, 3), lambda i: (i, 0)),
            pl.BlockSpec((NB, WA), lambda i: (i, 0)),
            pl.BlockSpec((NB, WA), lambda i: (i, 0)),
            pl.BlockSpec((NB, WA), lambda i: (i, 0)),
            pl.BlockSpec((NB, WA), lambda i: (i, 0)),
            _full((2 * F, H)), _full((1, H)), _full((H, F)), _full((1, F)),
            _full((F, H)), _full((1, H)), _full((H, 1)), _full((1, 1)),
        ],
        out_specs=[pl.BlockSpec((NB, 8), lambda i: (i, 0))],
        out_shape=[jax.ShapeDtypeStruct((N, 8), _f32)],
    )(h, x, a1, a2, d1, d2, wn1, bn1, wn2, bn2, wc1, bc1, wc2, bc2)[0]


def _tc_mu(parts2):
    def body(p_r, mu_r):
        p = p_r[...]
        sTot = p[:B, :] + p[B:, :]
        cnt = jnp.maximum(sTot[:, 7:8], 1.0)
        mu_r[...] = sTot[:, 3:6] - sTot[:, :3] * (sTot[:, 6:7] / cnt)

    return pl.pallas_call(
        body,
        in_specs=[_full((2 * B, 8))],
        out_specs=pl.BlockSpec((B, 3), lambda: (0, 0)),
        out_shape=jax.ShapeDtypeStruct((B, 3), _f32),
    )(parts2)


def kernel(z, pos, edge_index, batch, params):
    row2d = edge_index[0].astype(jnp.int32).reshape(NCH_E, CK)
    col2d = edge_index[1].astype(jnp.int32).reshape(NCH_E, CK)
    batch2d = batch.astype(jnp.int32).reshape(NCH_N, CKN)
    z2 = z.astype(jnp.int32).reshape(N, 1)

    embP = jnp.pad(params["emb"], ((0, 128 - params["emb"].shape[0]), (0, 0)))
    zf = jnp.zeros((N, WA), _f32)
    zb = jnp.zeros((B, 8), _f32)

    hch = NCH_E // 2
    rowA, rowB = row2d[:hch], row2d[hch:]
    colA, colB = col2d[:hch], col2d[hch:]

    def e1_split(p):
        W1 = p["e1"]["W"]
        return (W1[:F], W1[F:2 * F], p["e1"]["b"].reshape(1, H),
                W1[2 * F:2 * F + 1])

    layers = params["layers"]
    W1a, W1b, b1, w1c = e1_split(layers[0])
    h, tabR, tabC = _tc_embed_prep(z2, pos, embP, W1a, W1b, b1)
    x = pos

    for li, p in enumerate(layers):
        eW, eb = p["e2"]["W"], p["e2"]["b"].reshape(1, H)
        cW, cb = p["c"]["W"], p["c"]["b"].reshape(1, 1)
        grA, gcA = _sc_gather(tabR, tabC, rowA, colA)
        grB, gcB = _sc_gather(tabR, tabC, rowB, colB)
        vvA = _tc_edge(grA, gcA, eW, eb, cW, cb, w1c)
        vvB = _tc_edge(grB, gcB, eW, eb, cW, cb, w1c)
        a1p, a2p, d1p, d2p = _sc_scatter(rowA, vvA, zf, zf, zf, zf)
        a1, a2, d1, d2 = _sc_scatter(rowB, vvB, a1p, a2p, d1p, d2p)
        wn1 = p["n1"]["W"]
        bn1 = p["n1"]["b"].reshape(1, H)
        wn2 = p["n2"]["W"]
        bn2 = p["n2"]["b"].reshape(1, F)
        if li + 1 < len(layers):
            W1a, W1b, b1, w1c = e1_split(layers[li + 1])
            h, x, tabR, tabC = _tc_update_prep(
                h, x, a1, a2, d1, d2, wn1, bn1, wn2, bn2, W1a, W1b, b1)
        else:
            ch = params["charge"]
            nodevec = _tc_head(
                h, x, a1, a2, d1, d2, wn1, bn1, wn2, bn2,
                ch["l1"]["W"], ch["l1"]["b"].reshape(1, H),
                ch["l2"]["W"], ch["l2"]["b"].reshape(1, 1))

    parts = _sc_batchsum(batch2d, nodevec, zb)
    return _tc_mu(parts.reshape(2 * B, 8))
